# baseline (speedup 1.0000x reference)
import jax
import jax.numpy as jnp
from jax.experimental import pallas as pl
from jax.experimental.pallas import tpu as pltpu

TILE_B = 512
N_CHUNK = 4


def _mlp_fused_kernel(x_ref, w1_hbm, b1_ref, w2_hbm, b2_ref, o_ref,
                      w1_v, w2_v, sems):
    j = pl.program_id(0)
    H = w1_v.shape[1]
    ch = H // N_CHUNK

    def _w1_copy(c):
        return pltpu.make_async_copy(
            w1_hbm.at[:, pl.ds(c * ch, ch)],
            w1_v.at[:, pl.ds(c * ch, ch)],
            sems.at[c])

    def _w2_copy():
        return pltpu.make_async_copy(w2_hbm, w2_v, sems.at[N_CHUNK])

    @pl.when(j == 0)
    def _fetch_weights():
        for c in range(N_CHUNK):
            _w1_copy(c).start()
        _w2_copy().start()

    x = x_ref[...]
    hcs = []
    for c in range(N_CHUNK):
        @pl.when(j == 0)
        def _wait_chunk(c=c):
            _w1_copy(c).wait()
        hc = jnp.dot(x, w1_v[:, c * ch:(c + 1) * ch],
                     preferred_element_type=jnp.float32)
        hcs.append(jnp.maximum(hc + b1_ref[:, c * ch:(c + 1) * ch], 0.0))
    h = jnp.concatenate(hcs, axis=1)

    @pl.when(j == 0)
    def _wait_w2():
        _w2_copy().wait()

    out = jnp.dot(h, w2_v[...], preferred_element_type=jnp.float32)
    o_ref[...] = out + b2_ref[...]


@jax.jit
def kernel(x, w1t, b1r, w2t, b2r):
    B, I = x.shape
    H = w1t.shape[1]
    grid = (B // TILE_B,)

    flops = 4 * B * I * H
    bytes_accessed = 4 * (x.size + B * I + w1t.size + w2t.size)

    return pl.pallas_call(
        _mlp_fused_kernel,
        out_shape=jax.ShapeDtypeStruct((B, I), x.dtype),
        grid=grid,
        in_specs=[
            pl.BlockSpec((TILE_B, I), lambda i: (i, 0)),
            pl.BlockSpec(memory_space=pl.ANY),
            pl.BlockSpec((1, H), lambda i: (0, 0)),
            pl.BlockSpec(memory_space=pl.ANY),
            pl.BlockSpec((1, I), lambda i: (0, 0)),
        ],
        out_specs=pl.BlockSpec((TILE_B, I), lambda i: (i, 0)),
        scratch_shapes=[
            pltpu.VMEM((I, H), jnp.float32),
            pltpu.VMEM((H, I), jnp.float32),
            pltpu.SemaphoreType.DMA((N_CHUNK + 1,)),
        ],
        compiler_params=pltpu.CompilerParams(
            dimension_semantics=("arbitrary",),
            vmem_limit_bytes=64 * 1024 * 1024,
        ),
        cost_estimate=pl.CostEstimate(
            flops=flops, transcendentals=0, bytes_accessed=bytes_accessed),
    )(x, w1t, b1r, w2t, b2r)


# speedup vs baseline: 1.1062x; 1.1062x over previous
import jax
import jax.numpy as jnp
from jax.experimental import pallas as pl
from jax.experimental.pallas import tpu as pltpu

TILE_B = 1024


def _mlp_fused_kernel(x_ref, w1_hbm, b1_ref, w2_hbm, b2_ref, o_ref,
                      w1_v, w2_v, sems):
    j = pl.program_id(0)

    @pl.when(j == 0)
    def _fetch_weights():
        pltpu.make_async_copy(w1_hbm, w1_v, sems.at[0]).start()
        pltpu.make_async_copy(w1_hbm, w1_v, sems.at[0]).wait()
        pltpu.make_async_copy(w2_hbm, w2_v, sems.at[1]).start()

    h = jnp.dot(x_ref[...], w1_v[...], preferred_element_type=jnp.float32)
    h = jnp.maximum(h + b1_ref[...], 0.0)

    @pl.when(j == 0)
    def _wait_w2():
        pltpu.make_async_copy(w2_hbm, w2_v, sems.at[1]).wait()

    out = jnp.dot(h, w2_v[...], preferred_element_type=jnp.float32)
    o_ref[...] = out + b2_ref[...]


@jax.jit
def kernel(x, w1t, b1r, w2t, b2r):
    B, I = x.shape
    H = w1t.shape[1]
    grid = (B // TILE_B,)

    flops = 4 * B * I * H
    bytes_accessed = 4 * (x.size + B * I + w1t.size + w2t.size)

    return pl.pallas_call(
        _mlp_fused_kernel,
        out_shape=jax.ShapeDtypeStruct((B, I), x.dtype),
        grid=grid,
        in_specs=[
            pl.BlockSpec((TILE_B, I), lambda i: (i, 0)),
            pl.BlockSpec(memory_space=pl.ANY),
            pl.BlockSpec((1, H), lambda i: (0, 0)),
            pl.BlockSpec(memory_space=pl.ANY),
            pl.BlockSpec((1, I), lambda i: (0, 0)),
        ],
        out_specs=pl.BlockSpec((TILE_B, I), lambda i: (i, 0)),
        scratch_shapes=[
            pltpu.VMEM((I, H), jnp.float32),
            pltpu.VMEM((H, I), jnp.float32),
            pltpu.SemaphoreType.DMA((2,)),
        ],
        compiler_params=pltpu.CompilerParams(
            dimension_semantics=("arbitrary",),
            vmem_limit_bytes=64 * 1024 * 1024,
        ),
        cost_estimate=pl.CostEstimate(
            flops=flops, transcendentals=0, bytes_accessed=bytes_accessed),
    )(x, w1t, b1r, w2t, b2r)
